# baseline (speedup 1.0000x reference)
import functools

import jax
import jax.numpy as jnp
from jax import lax
from jax.experimental import pallas as pl
from jax.experimental.pallas import tpu as pltpu
from jax.experimental.pallas import tpu_sc as plsc

N = 10000
NPAD = 10016
E = 320000
IN_F = 128
H_F = 256
C_F = 64

NC = 2
NS = 16
NW = NC * NS
CHUNK = 112
DEG_B = 4
K = -(-(-(-E // (NW * CHUNK))) // 4) * 4
E_PAD = NW * K * CHUNK
RPT = NPAD // NS


def _mesh():
    return plsc.VectorSubcoreMesh(
        core_axis_name="c", subcore_axis_name="s", num_cores=NC, num_subcores=NS
    )


_sc_cache = {}


def _deg_body(srcp_h, dstp_h, ones_h, z16_h, outs_h, outd_h,
              sidx, didx, ones_v, acc, sems):
    c = lax.axis_index("c")
    s = lax.axis_index("s")
    wid = s * NC + c
    pltpu.sync_copy(srcp_h.at[wid], sidx)
    pltpu.sync_copy(dstp_h.at[wid], didx)
    pltpu.sync_copy(ones_h, ones_v)

    for idx, out_h in ((sidx, outs_h), (didx, outd_h)):
        pltpu.sync_copy(z16_h.at[pl.ds(s * RPT, RPT)], acc.at[pl.ds(s * RPT, RPT)])
        plsc.subcore_barrier()

        def body(g, carry, idx=idx):
            for b in range(DEG_B):
                pltpu.async_copy(ones_v, acc.at[idx.at[g * DEG_B + b]],
                                 sems.at[b], add=True)
            for b in range(DEG_B):
                pltpu.make_async_copy(ones_v, acc.at[idx.at[g * DEG_B + b]],
                                      sems.at[b]).wait()
            return carry

        lax.fori_loop(0, K // DEG_B, body, 0)
        plsc.subcore_barrier()
        pltpu.sync_copy(acc.at[pl.ds(s * RPT, RPT)], out_h.at[c, pl.ds(s * RPT, RPT)])
        plsc.subcore_barrier()


def _deg_kernel(srcp, dstp, ones16, z16):
    if "deg" not in _sc_cache:
        _sc_cache["deg"] = functools.partial(
            pl.kernel,
            out_type=[
                jax.ShapeDtypeStruct((NC, NPAD, 16), jnp.float32),
                jax.ShapeDtypeStruct((NC, NPAD, 16), jnp.float32),
            ],
            mesh=_mesh(),
            scratch_types=[
                pltpu.VMEM((K, CHUNK), jnp.int32),
                pltpu.VMEM((K, CHUNK), jnp.int32),
                pltpu.VMEM((CHUNK, 16), jnp.float32),
                pltpu.VMEM_SHARED((NPAD, 16), jnp.float32),
                pltpu.SemaphoreType.DMA((DEG_B,)),
            ],
            compiler_params=pltpu.CompilerParams(use_tc_tiling_on_sc=False),
        )(_deg_body)
    return _sc_cache["deg"](srcp, dstp, ones16, z16)


def _make_agg(D):
    NBUF = 2 if D == IN_F else 4

    def agg(tab_h, srcp_h, dstp_h, z_h, out_h, sidx, didx, *rest):
        rows = rest[:NBUF]
        acc, sems = rest[NBUF], rest[NBUF + 1]
        c = lax.axis_index("c")
        s = lax.axis_index("s")
        wid = s * NC + c
        pltpu.sync_copy(z_h.at[pl.ds(s * RPT, RPT)], acc.at[pl.ds(s * RPT, RPT)])
        pltpu.sync_copy(srcp_h.at[wid], sidx)
        pltpu.sync_copy(dstp_h.at[wid], didx)
        plsc.subcore_barrier()

        for b in range(NBUF):
            pltpu.async_copy(tab_h.at[sidx.at[b]], rows[b], sems.at[b])

        def outer(g, carry):
            for b in range(NBUF):
                j = g * NBUF + b
                pltpu.make_async_copy(
                    tab_h.at[sidx.at[j]], rows[b], sems.at[b]).wait()
                pltpu.sync_copy(rows[b], acc.at[didx.at[j]], add=True)

                @pl.when(g < K // NBUF - 1)
                def _prefetch():
                    pltpu.async_copy(
                        tab_h.at[sidx.at[j + NBUF]], rows[b], sems.at[b])
            return carry

        lax.fori_loop(0, K // NBUF, outer, 0)
        plsc.subcore_barrier()
        pltpu.sync_copy(acc.at[pl.ds(s * RPT, RPT)], out_h.at[c, pl.ds(s * RPT, RPT)])

    def call(tab, srcp, dstp, z):
        key = ("agg", D)
        if key not in _sc_cache:
            _sc_cache[key] = functools.partial(
                pl.kernel,
                out_type=jax.ShapeDtypeStruct((NC, NPAD, D), jnp.float32),
                mesh=_mesh(),
                scratch_types=[
                    pltpu.VMEM((K, CHUNK), jnp.int32),
                    pltpu.VMEM((K, CHUNK), jnp.int32),
                    *[pltpu.VMEM((CHUNK, D), jnp.float32) for _ in range(NBUF)],
                    pltpu.VMEM_SHARED((NPAD, D), jnp.float32),
                    pltpu.SemaphoreType.DMA((NBUF,)),
                ],
                compiler_params=pltpu.CompilerParams(use_tc_tiling_on_sc=False),
            )(agg)
        return _sc_cache[key](tab, srcp, dstp, z)

    return call


_agg128 = _make_agg(IN_F)
_agg64 = _make_agg(C_F)


_RB = 2504
_RF = 2000


def _feat_body(x_ref, ds_ref, o_ref):
    deg = ds_ref[0, :, 0:1] + ds_ref[1, :, 0:1]
    o_ref[...] = x_ref[...] * lax.rsqrt(jnp.maximum(deg, 1.0))


def _tc_feat(x_pad, deg_src):
    return pl.pallas_call(
        _feat_body,
        grid=(NPAD // _RB,),
        in_specs=[
            pl.BlockSpec((_RB, IN_F), lambda i: (i, 0)),
            pl.BlockSpec((NC, _RB, 16), lambda i: (0, i, 0)),
        ],
        out_specs=pl.BlockSpec((_RB, IN_F), lambda i: (i, 0)),
        out_shape=jax.ShapeDtypeStruct((NPAD, IN_F), jnp.float32),
    )(x_pad, deg_src)


def _mid_body(p_ref, dd_ref, ds_ref, w1_ref, b1_ref, w2_ref, o_ref):
    agg = p_ref[0] + p_ref[1]
    rin = lax.rsqrt(jnp.maximum(dd_ref[0, :, 0:1] + dd_ref[1, :, 0:1], 1.0))
    h = jnp.dot(agg * rin, w1_ref[...], preferred_element_type=jnp.float32)
    h = jnp.maximum(h + b1_ref[...], 0.0)
    rout = lax.rsqrt(jnp.maximum(ds_ref[0, :, 0:1] + ds_ref[1, :, 0:1], 1.0))
    o_ref[...] = jnp.dot(h * rout, w2_ref[...], preferred_element_type=jnp.float32)


def _tc_mid(parts1, deg_dst, deg_src, W1, b1, W2):
    return pl.pallas_call(
        _mid_body,
        grid=(NPAD // _RB,),
        in_specs=[
            pl.BlockSpec((NC, _RB, IN_F), lambda i: (0, i, 0)),
            pl.BlockSpec((NC, _RB, 16), lambda i: (0, i, 0)),
            pl.BlockSpec((NC, _RB, 16), lambda i: (0, i, 0)),
            pl.BlockSpec((IN_F, H_F), lambda i: (0, 0)),
            pl.BlockSpec((1, H_F), lambda i: (0, 0)),
            pl.BlockSpec((H_F, C_F), lambda i: (0, 0)),
        ],
        out_specs=pl.BlockSpec((_RB, C_F), lambda i: (i, 0)),
        out_shape=jax.ShapeDtypeStruct((NPAD, C_F), jnp.float32),
    )(parts1, deg_dst, deg_src, W1, b1.reshape(1, H_F), W2)


def _fin_body(p_ref, dd_ref, x_ref, b2_ref, h2_ref, gm_ref):
    i = pl.program_id(0)
    rin = lax.rsqrt(jnp.maximum(dd_ref[0, :, 0:1] + dd_ref[1, :, 0:1], 1.0))
    h2_ref[...] = (p_ref[0] + p_ref[1]) * rin + b2_ref[...]
    bm = jnp.max(x_ref[...], axis=0, keepdims=True)

    @pl.when(i == 0)
    def _init():
        gm_ref[...] = bm

    @pl.when(i != 0)
    def _acc():
        gm_ref[...] = jnp.maximum(gm_ref[...], bm)


def _tc_fin(parts2, deg_dst, x, b2):
    return pl.pallas_call(
        _fin_body,
        grid=(N // _RF,),
        in_specs=[
            pl.BlockSpec((NC, _RF, C_F), lambda i: (0, i, 0)),
            pl.BlockSpec((NC, _RF, 16), lambda i: (0, i, 0)),
            pl.BlockSpec((_RF, IN_F), lambda i: (i, 0)),
            pl.BlockSpec((1, C_F), lambda i: (0, 0)),
        ],
        out_specs=[
            pl.BlockSpec((_RF, C_F), lambda i: (i, 0)),
            pl.BlockSpec((1, IN_F), lambda i: (0, 0)),
        ],
        out_shape=[
            jax.ShapeDtypeStruct((N, C_F), jnp.float32),
            jax.ShapeDtypeStruct((1, IN_F), jnp.float32),
        ],
    )(parts2, deg_dst, x, b2.reshape(1, C_F))


def kernel(x, edge_index, W1, b1, W2, b2):
    pad = E_PAD - E
    srcp = jnp.concatenate(
        [edge_index[0], jnp.full((pad,), N, jnp.int32)]).reshape(NW, K, CHUNK)
    dstp = jnp.concatenate(
        [edge_index[1], jnp.full((pad,), N, jnp.int32)]).reshape(NW, K, CHUNK)
    x_pad = jnp.pad(x, ((0, NPAD - N), (0, 0)))
    ones16 = jnp.ones((CHUNK, 16), jnp.float32)
    z16 = jnp.zeros((NPAD, 16), jnp.float32)
    z128 = jnp.zeros((NPAD, IN_F), jnp.float32)
    z64 = jnp.zeros((NPAD, C_F), jnp.float32)

    deg_src, deg_dst = _deg_kernel(srcp, dstp, ones16, z16)
    feat1 = _tc_feat(x_pad, deg_src)
    parts1 = _agg128(feat1, srcp, dstp, z128)
    feat2 = _tc_mid(parts1, deg_dst, deg_src, W1, b1, W2)
    parts2 = _agg64(feat2, srcp, dstp, z64)
    h2, graph_max = _tc_fin(parts2, deg_dst, x, b2)
    return (graph_max, h2)


# speedup vs baseline: 3.7221x; 3.7221x over previous
import functools

import jax
import jax.numpy as jnp
from jax import lax
from jax.experimental import pallas as pl
from jax.experimental.pallas import tpu as pltpu
from jax.experimental.pallas import tpu_sc as plsc

N = 10000
NPAD = 10016
E = 320000
IN_F = 128
H_F = 256
C_F = 64

NC = 2
NS = 16
NW = NC * NS
CHUNK = 100
DEG_B = 4
K = E // (NW * CHUNK)
RPT = NPAD // NS


def _mesh():
    return plsc.VectorSubcoreMesh(
        core_axis_name="c", subcore_axis_name="s", num_cores=NC, num_subcores=NS
    )


_sc_cache = {}


def _deg_body(srcp_h, dstp_h, ones_h, z16_h, outs_h, outd_h,
              sidx, didx, ones_v, acc, sems):
    c = lax.axis_index("c")
    s = lax.axis_index("s")
    wid = s * NC + c
    pltpu.sync_copy(srcp_h.at[wid], sidx)
    pltpu.sync_copy(dstp_h.at[wid], didx)
    pltpu.sync_copy(ones_h, ones_v)

    for idx, out_h in ((sidx, outs_h), (didx, outd_h)):
        pltpu.sync_copy(z16_h.at[pl.ds(s * RPT, RPT)], acc.at[pl.ds(s * RPT, RPT)])
        plsc.subcore_barrier()

        def body(g, carry, idx=idx):
            for b in range(DEG_B):
                pltpu.async_copy(ones_v, acc.at[idx.at[g * DEG_B + b]],
                                 sems.at[b], add=True)
            for b in range(DEG_B):
                pltpu.make_async_copy(ones_v, acc.at[idx.at[g * DEG_B + b]],
                                      sems.at[b]).wait()
            return carry

        lax.fori_loop(0, K // DEG_B, body, 0)
        plsc.subcore_barrier()
        pltpu.sync_copy(acc.at[pl.ds(s * RPT, RPT)], out_h.at[c, pl.ds(s * RPT, RPT)])
        plsc.subcore_barrier()


def _deg_kernel(srcp, dstp, ones16, z16):
    if "deg" not in _sc_cache:
        _sc_cache["deg"] = functools.partial(
            pl.kernel,
            out_type=[
                jax.ShapeDtypeStruct((NC, NPAD, 16), jnp.float32),
                jax.ShapeDtypeStruct((NC, NPAD, 16), jnp.float32),
            ],
            mesh=_mesh(),
            scratch_types=[
                pltpu.VMEM((K, CHUNK), jnp.int32),
                pltpu.VMEM((K, CHUNK), jnp.int32),
                pltpu.VMEM((CHUNK, 16), jnp.float32),
                pltpu.VMEM_SHARED((NPAD, 16), jnp.float32),
                pltpu.SemaphoreType.DMA((DEG_B,)),
            ],
            compiler_params=pltpu.CompilerParams(use_tc_tiling_on_sc=False),
        )(_deg_body)
    return _sc_cache["deg"](srcp, dstp, ones16, z16)


def _make_agg(D):
    NBUF = 2 if D == IN_F else 4

    def agg(tab_h, srcp_h, dstp_h, z_h, out_h, sidx, didx, *rest):
        rows = rest[:NBUF]
        acc, sems = rest[NBUF], rest[NBUF + 1]
        c = lax.axis_index("c")
        s = lax.axis_index("s")
        wid = s * NC + c
        pltpu.sync_copy(z_h.at[pl.ds(s * RPT, RPT)], acc.at[pl.ds(s * RPT, RPT)])
        pltpu.sync_copy(srcp_h.at[wid], sidx)
        pltpu.sync_copy(dstp_h.at[wid], didx)
        plsc.subcore_barrier()

        for b in range(NBUF):
            pltpu.async_copy(tab_h.at[sidx.at[b]], rows[b], sems.at[b])

        def outer(g, carry):
            for b in range(NBUF):
                j = g * NBUF + b
                pltpu.make_async_copy(
                    tab_h.at[sidx.at[j]], rows[b], sems.at[b]).wait()
                pltpu.sync_copy(rows[b], acc.at[didx.at[j]], add=True)

                @pl.when(g < K // NBUF - 1)
                def _prefetch():
                    pltpu.async_copy(
                        tab_h.at[sidx.at[j + NBUF]], rows[b], sems.at[b])
            return carry

        lax.fori_loop(0, K // NBUF, outer, 0)
        plsc.subcore_barrier()
        pltpu.sync_copy(acc.at[pl.ds(s * RPT, RPT)], out_h.at[c, pl.ds(s * RPT, RPT)])

    def call(tab, srcp, dstp, z):
        key = ("agg", D)
        if key not in _sc_cache:
            _sc_cache[key] = functools.partial(
                pl.kernel,
                out_type=jax.ShapeDtypeStruct((NC, NPAD, D), jnp.float32),
                mesh=_mesh(),
                scratch_types=[
                    pltpu.VMEM((K, CHUNK), jnp.int32),
                    pltpu.VMEM((K, CHUNK), jnp.int32),
                    *[pltpu.VMEM((CHUNK, D), jnp.float32) for _ in range(NBUF)],
                    pltpu.VMEM_SHARED((NPAD, D), jnp.float32),
                    pltpu.SemaphoreType.DMA((NBUF,)),
                ],
                compiler_params=pltpu.CompilerParams(use_tc_tiling_on_sc=False),
            )(agg)
        return _sc_cache[key](tab, srcp, dstp, z)

    return call


_agg128 = _make_agg(IN_F)
_agg64 = _make_agg(C_F)


_RB = 2504
_RF = 2000


def _feat_body(x_ref, ds_ref, o_ref):
    deg = ds_ref[0, :, 0:1] + ds_ref[1, :, 0:1]
    o_ref[...] = x_ref[...] * lax.rsqrt(jnp.maximum(deg, 1.0))


def _tc_feat(x_pad, deg_src):
    return pl.pallas_call(
        _feat_body,
        grid=(NPAD // _RB,),
        in_specs=[
            pl.BlockSpec((_RB, IN_F), lambda i: (i, 0)),
            pl.BlockSpec((NC, _RB, 16), lambda i: (0, i, 0)),
        ],
        out_specs=pl.BlockSpec((_RB, IN_F), lambda i: (i, 0)),
        out_shape=jax.ShapeDtypeStruct((NPAD, IN_F), jnp.float32),
    )(x_pad, deg_src)


def _mid_body(p_ref, dd_ref, ds_ref, w1_ref, b1_ref, w2_ref, o_ref):
    agg = p_ref[0] + p_ref[1]
    rin = lax.rsqrt(jnp.maximum(dd_ref[0, :, 0:1] + dd_ref[1, :, 0:1], 1.0))
    h = jnp.dot(agg * rin, w1_ref[...], preferred_element_type=jnp.float32)
    h = jnp.maximum(h + b1_ref[...], 0.0)
    rout = lax.rsqrt(jnp.maximum(ds_ref[0, :, 0:1] + ds_ref[1, :, 0:1], 1.0))
    o_ref[...] = jnp.dot(h * rout, w2_ref[...], preferred_element_type=jnp.float32)


def _tc_mid(parts1, deg_dst, deg_src, W1, b1, W2):
    return pl.pallas_call(
        _mid_body,
        grid=(NPAD // _RB,),
        in_specs=[
            pl.BlockSpec((NC, _RB, IN_F), lambda i: (0, i, 0)),
            pl.BlockSpec((NC, _RB, 16), lambda i: (0, i, 0)),
            pl.BlockSpec((NC, _RB, 16), lambda i: (0, i, 0)),
            pl.BlockSpec((IN_F, H_F), lambda i: (0, 0)),
            pl.BlockSpec((1, H_F), lambda i: (0, 0)),
            pl.BlockSpec((H_F, C_F), lambda i: (0, 0)),
        ],
        out_specs=pl.BlockSpec((_RB, C_F), lambda i: (i, 0)),
        out_shape=jax.ShapeDtypeStruct((NPAD, C_F), jnp.float32),
    )(parts1, deg_dst, deg_src, W1, b1.reshape(1, H_F), W2)


def _fin_body(p_ref, dd_ref, x_ref, b2_ref, h2_ref, gm_ref):
    i = pl.program_id(0)
    rin = lax.rsqrt(jnp.maximum(dd_ref[0, :, 0:1] + dd_ref[1, :, 0:1], 1.0))
    h2_ref[...] = (p_ref[0] + p_ref[1]) * rin + b2_ref[...]
    bm = jnp.max(x_ref[...], axis=0, keepdims=True)

    @pl.when(i == 0)
    def _init():
        gm_ref[...] = bm

    @pl.when(i != 0)
    def _acc():
        gm_ref[...] = jnp.maximum(gm_ref[...], bm)


def _tc_fin(parts2, deg_dst, x, b2):
    return pl.pallas_call(
        _fin_body,
        grid=(N // _RF,),
        in_specs=[
            pl.BlockSpec((NC, _RF, C_F), lambda i: (0, i, 0)),
            pl.BlockSpec((NC, _RF, 16), lambda i: (0, i, 0)),
            pl.BlockSpec((_RF, IN_F), lambda i: (i, 0)),
            pl.BlockSpec((1, C_F), lambda i: (0, 0)),
        ],
        out_specs=[
            pl.BlockSpec((_RF, C_F), lambda i: (i, 0)),
            pl.BlockSpec((1, IN_F), lambda i: (0, 0)),
        ],
        out_shape=[
            jax.ShapeDtypeStruct((N, C_F), jnp.float32),
            jax.ShapeDtypeStruct((1, IN_F), jnp.float32),
        ],
    )(parts2, deg_dst, x, b2.reshape(1, C_F))


def kernel(x, edge_index, W1, b1, W2, b2):
    srcp = edge_index[0].reshape(NW, K, CHUNK)
    dstp = edge_index[1].reshape(NW, K, CHUNK)
    x_pad = jnp.pad(x, ((0, NPAD - N), (0, 0)))
    ones16 = jnp.ones((CHUNK, 16), jnp.float32)
    z16 = jnp.zeros((NPAD, 16), jnp.float32)
    z128 = jnp.zeros((NPAD, IN_F), jnp.float32)
    z64 = jnp.zeros((NPAD, C_F), jnp.float32)

    deg_src, deg_dst = _deg_kernel(srcp, dstp, ones16, z16)
    feat1 = _tc_feat(x_pad, deg_src)
    parts1 = _agg128(feat1, srcp, dstp, z128)
    feat2 = _tc_mid(parts1, deg_dst, deg_src, W1, b1, W2)
    parts2 = _agg64(feat2, srcp, dstp, z64)
    h2, graph_max = _tc_fin(parts2, deg_dst, x, b2)
    return (graph_max, h2)


# speedup vs baseline: 3.8956x; 1.0466x over previous
import functools

import jax
import jax.numpy as jnp
from jax import lax
from jax.experimental import pallas as pl
from jax.experimental.pallas import tpu as pltpu
from jax.experimental.pallas import tpu_sc as plsc

N = 10000
NPAD = 10016
E = 320000
IN_F = 128
H_F = 256
C_F = 64

NC = 2
NS = 16
NW = NC * NS
CHUNK = 100
DEG_B = 4
K = E // (NW * CHUNK)
RPT = NPAD // NS


def _mesh():
    return plsc.VectorSubcoreMesh(
        core_axis_name="c", subcore_axis_name="s", num_cores=NC, num_subcores=NS
    )


_sc_cache = {}


def _deg_body(srcp_h, dstp_h, ones_h, z16_h, outs_h, outd_h,
              sidx, didx, ones_v, acc, sems):
    c = lax.axis_index("c")
    s = lax.axis_index("s")
    wid = s * NC + c
    pltpu.sync_copy(srcp_h.at[wid], sidx)
    pltpu.sync_copy(dstp_h.at[wid], didx)
    pltpu.sync_copy(ones_h, ones_v)

    for idx, out_h in ((sidx, outs_h), (didx, outd_h)):
        pltpu.sync_copy(z16_h.at[pl.ds(s * RPT, RPT)], acc.at[pl.ds(s * RPT, RPT)])
        plsc.subcore_barrier()

        def body(g, carry, idx=idx):
            for b in range(DEG_B):
                pltpu.async_copy(ones_v, acc.at[idx.at[g * DEG_B + b]],
                                 sems.at[b], add=True)
            for b in range(DEG_B):
                pltpu.make_async_copy(ones_v, acc.at[idx.at[g * DEG_B + b]],
                                      sems.at[b]).wait()
            return carry

        lax.fori_loop(0, K // DEG_B, body, 0)
        plsc.subcore_barrier()
        pltpu.sync_copy(acc.at[pl.ds(s * RPT, RPT)], out_h.at[c, pl.ds(s * RPT, RPT)])
        plsc.subcore_barrier()


def _deg_kernel(srcp, dstp, ones16, z16):
    if "deg" not in _sc_cache:
        _sc_cache["deg"] = functools.partial(
            pl.kernel,
            out_type=[
                jax.ShapeDtypeStruct((NC, NPAD, 16), jnp.float32),
                jax.ShapeDtypeStruct((NC, NPAD, 16), jnp.float32),
            ],
            mesh=_mesh(),
            scratch_types=[
                pltpu.VMEM((K, CHUNK), jnp.int32),
                pltpu.VMEM((K, CHUNK), jnp.int32),
                pltpu.VMEM((CHUNK, 16), jnp.float32),
                pltpu.VMEM_SHARED((NPAD, 16), jnp.float32),
                pltpu.SemaphoreType.DMA((DEG_B,)),
            ],
            compiler_params=pltpu.CompilerParams(use_tc_tiling_on_sc=False),
        )(_deg_body)
    return _sc_cache["deg"](srcp, dstp, ones16, z16)


def _make_agg(D):
    NBUF = 4
    CH = 50 if D == IN_F else 100
    K_A = E // (NW * CH)

    def agg(tab_h, srcp_h, dstp_h, z_h, out_h, sidx, didx, *rest):
        rows = rest[:NBUF]
        acc, sems = rest[NBUF], rest[NBUF + 1]
        c = lax.axis_index("c")
        s = lax.axis_index("s")
        wid = s * NC + c
        pltpu.sync_copy(z_h.at[pl.ds(s * RPT, RPT)], acc.at[pl.ds(s * RPT, RPT)])
        pltpu.sync_copy(srcp_h.at[wid], sidx)
        pltpu.sync_copy(dstp_h.at[wid], didx)
        plsc.subcore_barrier()

        for b in range(NBUF):
            pltpu.async_copy(tab_h.at[sidx.at[b]], rows[b], sems.at[b])

        def outer(g, carry):
            for b in range(NBUF):
                j = g * NBUF + b
                pltpu.make_async_copy(
                    tab_h.at[sidx.at[j]], rows[b], sems.at[b]).wait()
                pltpu.sync_copy(rows[b], acc.at[didx.at[j]], add=True)

                @pl.when(g < K_A // NBUF - 1)
                def _prefetch():
                    pltpu.async_copy(
                        tab_h.at[sidx.at[j + NBUF]], rows[b], sems.at[b])
            return carry

        lax.fori_loop(0, K_A // NBUF, outer, 0)
        plsc.subcore_barrier()
        pltpu.sync_copy(acc.at[pl.ds(s * RPT, RPT)], out_h.at[c, pl.ds(s * RPT, RPT)])

    def call(tab, srcp, dstp, z):
        srcp = srcp.reshape(NW, K_A, CH)
        dstp = dstp.reshape(NW, K_A, CH)
        key = ("agg", D)
        if key not in _sc_cache:
            _sc_cache[key] = functools.partial(
                pl.kernel,
                out_type=jax.ShapeDtypeStruct((NC, NPAD, D), jnp.float32),
                mesh=_mesh(),
                scratch_types=[
                    pltpu.VMEM((K_A, CH), jnp.int32),
                    pltpu.VMEM((K_A, CH), jnp.int32),
                    *[pltpu.VMEM((CH, D), jnp.float32) for _ in range(NBUF)],
                    pltpu.VMEM_SHARED((NPAD, D), jnp.float32),
                    pltpu.SemaphoreType.DMA((NBUF,)),
                ],
                compiler_params=pltpu.CompilerParams(use_tc_tiling_on_sc=False),
            )(agg)
        return _sc_cache[key](tab, srcp, dstp, z)

    return call


_agg128 = _make_agg(IN_F)
_agg64 = _make_agg(C_F)


_RB = 2504
_RF = 2000


def _feat_body(x_ref, ds_ref, o_ref):
    deg = ds_ref[0, :, 0:1] + ds_ref[1, :, 0:1]
    o_ref[...] = x_ref[...] * lax.rsqrt(jnp.maximum(deg, 1.0))


def _tc_feat(x_pad, deg_src):
    return pl.pallas_call(
        _feat_body,
        grid=(NPAD // _RB,),
        in_specs=[
            pl.BlockSpec((_RB, IN_F), lambda i: (i, 0)),
            pl.BlockSpec((NC, _RB, 16), lambda i: (0, i, 0)),
        ],
        out_specs=pl.BlockSpec((_RB, IN_F), lambda i: (i, 0)),
        out_shape=jax.ShapeDtypeStruct((NPAD, IN_F), jnp.float32),
    )(x_pad, deg_src)


def _mid_body(p_ref, dd_ref, ds_ref, w1_ref, b1_ref, w2_ref, o_ref):
    agg = p_ref[0] + p_ref[1]
    rin = lax.rsqrt(jnp.maximum(dd_ref[0, :, 0:1] + dd_ref[1, :, 0:1], 1.0))
    h = jnp.dot(agg * rin, w1_ref[...], preferred_element_type=jnp.float32)
    h = jnp.maximum(h + b1_ref[...], 0.0)
    rout = lax.rsqrt(jnp.maximum(ds_ref[0, :, 0:1] + ds_ref[1, :, 0:1], 1.0))
    o_ref[...] = jnp.dot(h * rout, w2_ref[...], preferred_element_type=jnp.float32)


def _tc_mid(parts1, deg_dst, deg_src, W1, b1, W2):
    return pl.pallas_call(
        _mid_body,
        grid=(NPAD // _RB,),
        in_specs=[
            pl.BlockSpec((NC, _RB, IN_F), lambda i: (0, i, 0)),
            pl.BlockSpec((NC, _RB, 16), lambda i: (0, i, 0)),
            pl.BlockSpec((NC, _RB, 16), lambda i: (0, i, 0)),
            pl.BlockSpec((IN_F, H_F), lambda i: (0, 0)),
            pl.BlockSpec((1, H_F), lambda i: (0, 0)),
            pl.BlockSpec((H_F, C_F), lambda i: (0, 0)),
        ],
        out_specs=pl.BlockSpec((_RB, C_F), lambda i: (i, 0)),
        out_shape=jax.ShapeDtypeStruct((NPAD, C_F), jnp.float32),
    )(parts1, deg_dst, deg_src, W1, b1.reshape(1, H_F), W2)


def _fin_body(p_ref, dd_ref, x_ref, b2_ref, h2_ref, gm_ref):
    i = pl.program_id(0)
    rin = lax.rsqrt(jnp.maximum(dd_ref[0, :, 0:1] + dd_ref[1, :, 0:1], 1.0))
    h2_ref[...] = (p_ref[0] + p_ref[1]) * rin + b2_ref[...]
    bm = jnp.max(x_ref[...], axis=0, keepdims=True)

    @pl.when(i == 0)
    def _init():
        gm_ref[...] = bm

    @pl.when(i != 0)
    def _acc():
        gm_ref[...] = jnp.maximum(gm_ref[...], bm)


def _tc_fin(parts2, deg_dst, x, b2):
    return pl.pallas_call(
        _fin_body,
        grid=(N // _RF,),
        in_specs=[
            pl.BlockSpec((NC, _RF, C_F), lambda i: (0, i, 0)),
            pl.BlockSpec((NC, _RF, 16), lambda i: (0, i, 0)),
            pl.BlockSpec((_RF, IN_F), lambda i: (i, 0)),
            pl.BlockSpec((1, C_F), lambda i: (0, 0)),
        ],
        out_specs=[
            pl.BlockSpec((_RF, C_F), lambda i: (i, 0)),
            pl.BlockSpec((1, IN_F), lambda i: (0, 0)),
        ],
        out_shape=[
            jax.ShapeDtypeStruct((N, C_F), jnp.float32),
            jax.ShapeDtypeStruct((1, IN_F), jnp.float32),
        ],
    )(parts2, deg_dst, x, b2.reshape(1, C_F))


def kernel(x, edge_index, W1, b1, W2, b2):
    srcp = edge_index[0].reshape(NW, K, CHUNK)
    dstp = edge_index[1].reshape(NW, K, CHUNK)
    x_pad = jnp.pad(x, ((0, NPAD - N), (0, 0)))
    ones16 = jnp.ones((CHUNK, 16), jnp.float32)
    z16 = jnp.zeros((NPAD, 16), jnp.float32)
    z128 = jnp.zeros((NPAD, IN_F), jnp.float32)
    z64 = jnp.zeros((NPAD, C_F), jnp.float32)

    deg_src, deg_dst = _deg_kernel(srcp, dstp, ones16, z16)
    feat1 = _tc_feat(x_pad, deg_src)
    parts1 = _agg128(feat1, srcp, dstp, z128)
    feat2 = _tc_mid(parts1, deg_dst, deg_src, W1, b1, W2)
    parts2 = _agg64(feat2, srcp, dstp, z64)
    h2, graph_max = _tc_fin(parts2, deg_dst, x, b2)
    return (graph_max, h2)
